# Initial kernel scaffold; baseline (speedup 1.0000x reference)
import jax
import jax.numpy as jnp
from jax.experimental import pallas as pl


def kernel(input_ids, word_table, pos_table, gamma, beta):
    raise NotImplementedError("write your pallas kernel here")


# speedup vs baseline: 2.0724x; 2.0724x over previous
import functools

import jax
import jax.numpy as jnp
from jax import lax
from jax.experimental import pallas as pl
from jax.experimental.pallas import tpu as pltpu
from jax.experimental.pallas import tpu_sc as plsc

_EPS = 1e-12


def _take16(x, idx):
    return lax.gather(
        x, idx[:, None],
        lax.GatherDimensionNumbers(
            offset_dims=(), collapsed_slice_dims=(0,), start_index_map=(0,)),
        (1,), mode=lax.GatherScatterMode.PROMISE_IN_BOUNDS)


def _allsum16(x):
    lanes = lax.iota(jnp.int32, 16)
    for k in (1, 2, 4, 8):
        x = x + _take16(x, lanes ^ k)
    return x


def _rsqrt16(v):
    i = lax.bitcast_convert_type(v, jnp.int32)
    i = jnp.int32(0x5F3759DF) - lax.shift_right_logical(i, 1)
    y = lax.bitcast_convert_type(i, jnp.float32)
    for _ in range(3):
        y = y * (1.5 - 0.5 * v * y * y)
    return y


def kernel(input_ids, word_table, pos_table, gamma, beta):
    B, L = input_ids.shape
    V, D = word_table.shape
    N = B * L

    ids = input_ids.reshape(N)
    pos = pos_table[:L]

    info = plsc.get_sparse_core_info()
    NW = info.num_cores * info.num_subcores
    NC = info.num_cores

    rows_per_w = N // NW
    C = L
    chunks_per_w = rows_per_w // C
    C0 = 104
    C1 = C - C0

    mesh = plsc.VectorSubcoreMesh(core_axis_name="c", subcore_axis_name="s")

    @functools.partial(
        pl.kernel,
        mesh=mesh,
        compiler_params=pltpu.CompilerParams(use_tc_tiling_on_sc=False),
        out_type=jax.ShapeDtypeStruct((N, D), jnp.float32),
        scratch_types=[
            pltpu.VMEM((C,), jnp.int32),
            pltpu.VMEM((C, D), jnp.float32),
            pltpu.VMEM((C, D), jnp.float32),
            pltpu.VMEM((D,), jnp.float32),
            pltpu.VMEM((D,), jnp.float32),
            pltpu.SemaphoreType.DMA,
        ],
    )
    def k(ids_hbm, table_hbm, pos_hbm, g_hbm, b_hbm, out_hbm,
          idx_v, rows_v, pos_v, g_v, b_v, sem):
        wid = lax.axis_index("s") * NC + lax.axis_index("c")

        pltpu.sync_copy(pos_hbm, pos_v)
        pltpu.sync_copy(g_hbm, g_v)
        pltpu.sync_copy(b_hbm, b_v)

        gs = [g_v[pl.ds(16 * t, 16)] for t in range(4)]
        bs = [b_v[pl.ds(16 * t, 16)] for t in range(4)]

        def one_row(r):
            xs = [rows_v[r, pl.ds(16 * t, 16)] + pos_v[r, pl.ds(16 * t, 16)]
                  for t in range(4)]
            s = (xs[0] + xs[1]) + (xs[2] + xs[3])
            q = ((xs[0] * xs[0] + xs[1] * xs[1])
                 + (xs[2] * xs[2] + xs[3] * xs[3]))
            sv = _allsum16(s)
            qv = _allsum16(q)
            mean = sv * (1.0 / 64.0)
            var = qv * (1.0 / 64.0) - mean * mean
            rstd = _rsqrt16(var + _EPS)
            for t in range(4):
                rows_v[r, pl.ds(16 * t, 16)] = (xs[t] - mean) * (rstd * gs[t]) + bs[t]

        def chunk_body(c, carry):
            base = (wid * chunks_per_w + c) * C
            pltpu.sync_copy(ids_hbm.at[pl.ds(base, C)], idx_v)
            cp0 = pltpu.async_copy(
                table_hbm.at[idx_v.at[pl.ds(0, C0)]],
                rows_v.at[pl.ds(0, C0)], sem)
            cp1 = pltpu.async_copy(
                table_hbm.at[idx_v.at[pl.ds(C0, C1)]],
                rows_v.at[pl.ds(C0, C1)], sem)
            cp0.wait()
            cp1.wait()

            def row4(i, c2):
                for u in range(4):
                    one_row(i * 4 + u)
                return c2

            lax.fori_loop(0, C // 4, row4, 0)
            pltpu.sync_copy(rows_v, out_hbm.at[pl.ds(base, C)])
            return carry

        lax.fori_loop(0, chunks_per_w, chunk_body, 0)

    out = k(ids, word_table, pos, gamma, beta)
    return out.reshape(B, L, D)


# speedup vs baseline: 2.3566x; 1.1371x over previous
import functools

import jax
import jax.numpy as jnp
from jax import lax
from jax.experimental import pallas as pl
from jax.experimental.pallas import tpu as pltpu
from jax.experimental.pallas import tpu_sc as plsc

_EPS = 1e-12


def _take16(x, idx):
    return lax.gather(
        x, idx[:, None],
        lax.GatherDimensionNumbers(
            offset_dims=(), collapsed_slice_dims=(0,), start_index_map=(0,)),
        (1,), mode=lax.GatherScatterMode.PROMISE_IN_BOUNDS)


def _allsum16(x):
    lanes = lax.iota(jnp.int32, 16)
    for k in (1, 2, 4, 8):
        x = x + _take16(x, lanes ^ k)
    return x


def _rsqrt16(v):
    i = lax.bitcast_convert_type(v, jnp.int32)
    i = jnp.int32(0x5F3759DF) - lax.shift_right_logical(i, 1)
    y = lax.bitcast_convert_type(i, jnp.float32)
    for _ in range(3):
        y = y * (1.5 - 0.5 * v * y * y)
    return y


def kernel(input_ids, word_table, pos_table, gamma, beta):
    B, L = input_ids.shape
    V, D = word_table.shape
    N = B * L

    ids = input_ids.reshape(N)
    pos = pos_table[:L]

    info = plsc.get_sparse_core_info()
    NW = info.num_cores * info.num_subcores
    NC = info.num_cores

    rows_per_w = N // NW
    C = L
    chunks_per_w = rows_per_w // C
    C0 = 104
    C1 = C - C0

    mesh = plsc.VectorSubcoreMesh(core_axis_name="c", subcore_axis_name="s")

    @functools.partial(
        pl.kernel,
        mesh=mesh,
        compiler_params=pltpu.CompilerParams(use_tc_tiling_on_sc=False),
        out_type=jax.ShapeDtypeStruct((N, D), jnp.float32),
        scratch_types=[
            pltpu.VMEM((C,), jnp.int32),
            pltpu.VMEM((C,), jnp.int32),
            pltpu.VMEM((C, D), jnp.float32),
            pltpu.VMEM((C, D), jnp.float32),
            pltpu.VMEM((C, D), jnp.float32),
            pltpu.VMEM((D,), jnp.float32),
            pltpu.VMEM((D,), jnp.float32),
            pltpu.SemaphoreType.DMA,
            pltpu.SemaphoreType.DMA,
            pltpu.SemaphoreType.DMA,
            pltpu.SemaphoreType.DMA,
        ],
    )
    def k(ids_hbm, table_hbm, pos_hbm, g_hbm, b_hbm, out_hbm,
          idx0, idx1, rows0, rows1, pos_v, g_v, b_v,
          sem_g0, sem_g1, sem_i0, sem_i1):
        wid = lax.axis_index("s") * NC + lax.axis_index("c")

        pltpu.sync_copy(pos_hbm, pos_v)
        pltpu.sync_copy(g_hbm, g_v)
        pltpu.sync_copy(b_hbm, b_v)

        gs = [g_v[pl.ds(16 * t, 16)] for t in range(4)]
        bs = [b_v[pl.ds(16 * t, 16)] for t in range(4)]

        def one_row(rows_v, r):
            xs = [rows_v[r, pl.ds(16 * t, 16)] + pos_v[r, pl.ds(16 * t, 16)]
                  for t in range(4)]
            s = (xs[0] + xs[1]) + (xs[2] + xs[3])
            q = ((xs[0] * xs[0] + xs[1] * xs[1])
                 + (xs[2] * xs[2] + xs[3] * xs[3]))
            sv = _allsum16(s)
            qv = _allsum16(q)
            mean = sv * (1.0 / 64.0)
            var = qv * (1.0 / 64.0) - mean * mean
            rstd = _rsqrt16(var + _EPS)
            for t in range(4):
                rows_v[r, pl.ds(16 * t, 16)] = (xs[t] - mean) * (rstd * gs[t]) + bs[t]

        w_base = wid * chunks_per_w * C

        def issue_idx(c, idx_v, sem):
            pltpu.async_copy(ids_hbm.at[pl.ds(w_base + c * C, C)], idx_v, sem)

        def wait_idx(idx_v, sem):
            pltpu.make_async_copy(ids_hbm.at[pl.ds(w_base, C)], idx_v,
                                  sem).wait()

        def issue_gather(idx_v, rows_v, sem):
            pltpu.async_copy(table_hbm.at[idx_v.at[pl.ds(0, C0)]],
                             rows_v.at[pl.ds(0, C0)], sem)
            pltpu.async_copy(table_hbm.at[idx_v.at[pl.ds(C0, C1)]],
                             rows_v.at[pl.ds(C0, C1)], sem)

        def wait_gather(idx_v, rows_v, sem):
            pltpu.make_async_copy(table_hbm.at[idx_v.at[pl.ds(0, C0)]],
                                  rows_v.at[pl.ds(0, C0)], sem).wait()
            pltpu.make_async_copy(table_hbm.at[idx_v.at[pl.ds(C0, C1)]],
                                  rows_v.at[pl.ds(C0, C1)], sem).wait()

        def compute(rows_v):
            def row4(i, c2):
                for u in range(4):
                    one_row(rows_v, i * 4 + u)
                return c2

            lax.fori_loop(0, C // 4, row4, 0)

        pltpu.sync_copy(ids_hbm.at[pl.ds(w_base, C)], idx0)
        issue_gather(idx0, rows0, sem_g0)
        issue_idx(1, idx1, sem_i1)

        G = chunks_per_w // 2

        def body(g, carry):
            c = 2 * g
            not_last = g < G - 1

            wait_gather(idx0, rows0, sem_g0)
            wait_idx(idx1, sem_i1)
            issue_gather(idx1, rows1, sem_g1)

            @pl.when(not_last)
            def _():
                issue_idx(c + 2, idx0, sem_i0)

            compute(rows0)
            pltpu.sync_copy(rows0, out_hbm.at[pl.ds(w_base + c * C, C)])

            wait_gather(idx1, rows1, sem_g1)

            @pl.when(not_last)
            def _():
                wait_idx(idx0, sem_i0)
                issue_gather(idx0, rows0, sem_g0)
                issue_idx(c + 3, idx1, sem_i1)

            compute(rows1)
            pltpu.sync_copy(rows1, out_hbm.at[pl.ds(w_base + (c + 1) * C, C)])
            return carry

        lax.fori_loop(0, G, body, 0)

    out = k(ids, word_table, pos, gamma, beta)
    return out.reshape(B, L, D)
